# Initial kernel scaffold; baseline (speedup 1.0000x reference)
import jax
import jax.numpy as jnp
from jax.experimental import pallas as pl


def kernel(x, adj, W1, W2, W3, C, lw1, lb1, lw2, lb2, lw3, lb3):
    raise NotImplementedError("write your pallas kernel here")


# speedup vs baseline: 1.2232x; 1.2232x over previous
import functools

import jax
import jax.numpy as jnp
from jax.experimental import pallas as pl


def _elu(v):
    return jnp.where(v > 0, v, jnp.exp(jnp.minimum(v, 0.0)) - 1.0)


def _xw_kernel(x_ref, w_ref, o_ref):
    o_ref[...] = jnp.dot(x_ref[...], w_ref[...],
                         preferred_element_type=jnp.float32)


def _pass1_kernel(adj_ref, xw1_ref, w23_ref, g_ref):
    h = jnp.dot(adj_ref[...], xw1_ref[...], preferred_element_type=jnp.float32)
    h = jnp.maximum(h, 0.0)
    g_ref[...] = jnp.dot(h, w23_ref[...], preferred_element_type=jnp.float32)


def _pass2_kernel(adj_ref, g_ref, c_ref, lw1_ref, lb1_ref, lw2_ref, lb2_ref,
                  lw3_ref, lb3_ref, mu_ref, lv_ref, z_ref, label_ref, h2):
    acc = jnp.dot(adj_ref[...], g_ref[...], preferred_element_type=jnp.float32)
    mu = acc[:, :h2]
    mu_ref[...] = mu
    lv_ref[...] = acc[:, h2:]
    z = jnp.dot(mu, c_ref[...], preferred_element_type=jnp.float32)
    z_ref[...] = z
    h = _elu(jnp.dot(z, lw1_ref[...], preferred_element_type=jnp.float32)
             + lb1_ref[...])
    h = _elu(jnp.dot(h, lw2_ref[...], preferred_element_type=jnp.float32)
             + lb2_ref[...])
    label_ref[...] = (jnp.dot(h, lw3_ref[...],
                              preferred_element_type=jnp.float32)
                      + lb3_ref[...])


def _decoder_kernel(z_ref, zall_ref, o_ref):
    o_ref[...] = jax.lax.dot_general(
        z_ref[...], zall_ref[...],
        dimension_numbers=(((1,), (1,)), ((), ())),
        preferred_element_type=jnp.float32)


def kernel(x, adj, W1, W2, W3, C, lw1, lb1, lw2, lb2, lw3, lb3):
    n, d_in = x.shape
    h1 = W1.shape[1]
    h2 = W2.shape[1]
    w23 = jnp.concatenate([W2, W3], axis=1)
    lb1r = lb1.reshape(1, -1)
    lb2r = lb2.reshape(1, -1)
    lb3r = lb3.reshape(1, -1)

    bi = 200 if n % 200 == 0 else n
    ni = n // bi

    xw1 = pl.pallas_call(
        _xw_kernel,
        out_shape=jax.ShapeDtypeStruct((n, h1), jnp.float32),
    )(x, W1)

    g = pl.pallas_call(
        _pass1_kernel,
        grid=(ni,),
        in_specs=[
            pl.BlockSpec((bi, n), lambda i: (i, 0)),
            pl.BlockSpec((n, h1), lambda i: (0, 0)),
            pl.BlockSpec((h1, 2 * h2), lambda i: (0, 0)),
        ],
        out_specs=pl.BlockSpec((bi, 2 * h2), lambda i: (i, 0)),
        out_shape=jax.ShapeDtypeStruct((n, 2 * h2), jnp.float32),
    )(adj, xw1, w23)

    small = lambda a: pl.BlockSpec(a.shape, lambda i: (0,) * a.ndim)
    mu, logvar, z, label = pl.pallas_call(
        functools.partial(_pass2_kernel, h2=h2),
        grid=(ni,),
        in_specs=[
            pl.BlockSpec((bi, n), lambda i: (i, 0)),
            pl.BlockSpec((n, 2 * h2), lambda i: (0, 0)),
            small(C), small(lw1), small(lb1r), small(lw2), small(lb2r),
            small(lw3), small(lb3r),
        ],
        out_specs=[
            pl.BlockSpec((bi, h2), lambda i: (i, 0)),
            pl.BlockSpec((bi, h2), lambda i: (i, 0)),
            pl.BlockSpec((bi, h2), lambda i: (i, 0)),
            pl.BlockSpec((bi, d_in), lambda i: (i, 0)),
        ],
        out_shape=[
            jax.ShapeDtypeStruct((n, h2), jnp.float32),
            jax.ShapeDtypeStruct((n, h2), jnp.float32),
            jax.ShapeDtypeStruct((n, h2), jnp.float32),
            jax.ShapeDtypeStruct((n, d_in), jnp.float32),
        ],
    )(adj, g, C, lw1, lb1r, lw2, lb2r, lw3, lb3r)

    adj_rec = pl.pallas_call(
        _decoder_kernel,
        grid=(ni,),
        in_specs=[
            pl.BlockSpec((bi, h2), lambda i: (i, 0)),
            pl.BlockSpec((n, h2), lambda i: (0, 0)),
        ],
        out_specs=pl.BlockSpec((bi, n), lambda i: (i, 0)),
        out_shape=jax.ShapeDtypeStruct((n, n), jnp.float32),
    )(z, z)

    return (label, adj_rec, mu, logvar, mu, z)


# speedup vs baseline: 1.2762x; 1.0433x over previous
import functools

import jax
import jax.numpy as jnp
from jax.experimental import pallas as pl
from jax.experimental.pallas import tpu as pltpu


def _elu(v):
    return jnp.where(v > 0, v, jnp.exp(jnp.minimum(v, 0.0)) - 1.0)


def _pass1_kernel(x_ref, w1_ref, adj_ref, w23_ref, g_ref, xw1_ref):
    @pl.when(pl.program_id(0) == 0)
    def _():
        xw1_ref[...] = jnp.dot(x_ref[...], w1_ref[...],
                               preferred_element_type=jnp.float32)
    h = jnp.dot(adj_ref[...], xw1_ref[...], preferred_element_type=jnp.float32)
    h = jnp.maximum(h, 0.0)
    g_ref[...] = jnp.dot(h, w23_ref[...], preferred_element_type=jnp.float32)


def _pass2_kernel(adj_ref, g_ref, c_ref, lw1_ref, lb1_ref, lw2_ref, lb2_ref,
                  lw3_ref, lb3_ref, mu_ref, lv_ref, z_ref, label_ref, h2):
    acc = jnp.dot(adj_ref[...], g_ref[...], preferred_element_type=jnp.float32)
    mu = acc[:, :h2]
    mu_ref[...] = mu
    lv_ref[...] = acc[:, h2:]
    z = jnp.dot(mu, c_ref[...], preferred_element_type=jnp.float32)
    z_ref[...] = z
    h = _elu(jnp.dot(z, lw1_ref[...], preferred_element_type=jnp.float32)
             + lb1_ref[...])
    h = _elu(jnp.dot(h, lw2_ref[...], preferred_element_type=jnp.float32)
             + lb2_ref[...])
    label_ref[...] = (jnp.dot(h, lw3_ref[...],
                              preferred_element_type=jnp.float32)
                      + lb3_ref[...])


def _decoder_kernel(z_ref, zall_ref, o_ref):
    o_ref[...] = jax.lax.dot_general(
        z_ref[...], zall_ref[...],
        dimension_numbers=(((1,), (1,)), ((), ())),
        preferred_element_type=jnp.float32)


def kernel(x, adj, W1, W2, W3, C, lw1, lb1, lw2, lb2, lw3, lb3):
    n, d_in = x.shape
    h1 = W1.shape[1]
    h2 = W2.shape[1]
    w23 = jnp.concatenate([W2, W3], axis=1)
    lb1r = lb1.reshape(1, -1)
    lb2r = lb2.reshape(1, -1)
    lb3r = lb3.reshape(1, -1)

    bi = 400 if n % 400 == 0 else n
    ni = n // bi

    g = pl.pallas_call(
        _pass1_kernel,
        grid=(ni,),
        in_specs=[
            pl.BlockSpec((n, d_in), lambda i: (0, 0)),
            pl.BlockSpec((d_in, h1), lambda i: (0, 0)),
            pl.BlockSpec((bi, n), lambda i: (i, 0)),
            pl.BlockSpec((h1, 2 * h2), lambda i: (0, 0)),
        ],
        out_specs=pl.BlockSpec((bi, 2 * h2), lambda i: (i, 0)),
        out_shape=jax.ShapeDtypeStruct((n, 2 * h2), jnp.float32),
        scratch_shapes=[pltpu.VMEM((n, h1), jnp.float32)],
    )(x, W1, adj, w23)

    small = lambda a: pl.BlockSpec(a.shape, lambda i: (0,) * a.ndim)
    mu, logvar, z, label = pl.pallas_call(
        functools.partial(_pass2_kernel, h2=h2),
        grid=(ni,),
        in_specs=[
            pl.BlockSpec((bi, n), lambda i: (i, 0)),
            pl.BlockSpec((n, 2 * h2), lambda i: (0, 0)),
            small(C), small(lw1), small(lb1r), small(lw2), small(lb2r),
            small(lw3), small(lb3r),
        ],
        out_specs=[
            pl.BlockSpec((bi, h2), lambda i: (i, 0)),
            pl.BlockSpec((bi, h2), lambda i: (i, 0)),
            pl.BlockSpec((bi, h2), lambda i: (i, 0)),
            pl.BlockSpec((bi, d_in), lambda i: (i, 0)),
        ],
        out_shape=[
            jax.ShapeDtypeStruct((n, h2), jnp.float32),
            jax.ShapeDtypeStruct((n, h2), jnp.float32),
            jax.ShapeDtypeStruct((n, h2), jnp.float32),
            jax.ShapeDtypeStruct((n, d_in), jnp.float32),
        ],
    )(adj, g, C, lw1, lb1r, lw2, lb2r, lw3, lb3r)

    adj_rec = pl.pallas_call(
        _decoder_kernel,
        grid=(ni,),
        in_specs=[
            pl.BlockSpec((bi, h2), lambda i: (i, 0)),
            pl.BlockSpec((n, h2), lambda i: (0, 0)),
        ],
        out_specs=pl.BlockSpec((bi, n), lambda i: (i, 0)),
        out_shape=jax.ShapeDtypeStruct((n, n), jnp.float32),
    )(z, z)

    return (label, adj_rec, mu, logvar, mu, z)
